# baseline (speedup 1.0000x reference)
import functools

import jax
import jax.numpy as jnp
from jax import lax
from jax.experimental import pallas as pl
from jax.experimental.pallas import tpu as pltpu
from jax.experimental.pallas import tpu_sc as plsc

K = 16
NUM_RANKS = 31

B = 4
N = 4096
C = 256
BM = 256
NCHUNK = 32

NWORKERS = 32
ROWS = B * N
RPW = ROWS // NWORKERS
CANDCAP = N + 32
BIGI = 2**30


def _dist_block(q_ref, qt_ref, dist_ref, thr_ref):
    qb = q_ref[0]
    st = qt_ref[0]

    b2 = jnp.sum(qb * qb, axis=1, keepdims=True)
    a2 = jnp.sum(st * st, axis=0, keepdims=True)
    dot = jax.lax.dot_general(
        qb, st, (((1,), (0,)), ((), ())),
        preferred_element_type=jnp.float32)
    d2 = (a2 + b2) - 2.0 * dot
    dist = jnp.sqrt(jnp.maximum(d2, 1e-12))
    dist_ref[0] = dist

    w = N // NCHUNK
    thr = jnp.min(dist[:, :w], axis=1, keepdims=True)
    for c in range(1, NCHUNK):
        cm = jnp.min(dist[:, c * w:(c + 1) * w], axis=1, keepdims=True)
        thr = jnp.maximum(thr, cm)
    thr_ref[0] = thr


def _tc_stage(query):
    nb = query.shape[0]
    qt = jnp.swapaxes(query, 1, 2)
    return pl.pallas_call(
        _dist_block,
        grid=(nb, N // BM),
        in_specs=[
            pl.BlockSpec((1, BM, C), lambda b, i: (b, i, 0)),
            pl.BlockSpec((1, C, N), lambda b, i: (b, 0, 0)),
        ],
        out_specs=[
            pl.BlockSpec((1, BM, N), lambda b, i: (b, i, 0)),
            pl.BlockSpec((1, BM, 1), lambda b, i: (b, i, 0)),
        ],
        out_shape=[
            jax.ShapeDtypeStruct((nb, N, N), jnp.float32),
            jax.ShapeDtypeStruct((nb, N, 1), jnp.float32),
        ],
    )(query, qt)


def _lex_exchange(ak, ai, bk, bi):
    t = (ak < bk) | ((ak == bk) & (ai < bi))
    lok = jnp.where(t, ak, bk)
    loi = jnp.where(t, ai, bi)
    hik = jnp.where(t, bk, ak)
    hii = jnp.where(t, bi, ai)
    return lok, loi, hik, hii


def _rev(x):
    return lax.rev(x, (0,))


def _lex_min(ak, ai, bk, bi):
    t = (ak < bk) | ((ak == bk) & (ai < bi))
    return jnp.where(t, ak, bk), jnp.where(t, ai, bi)


def _merge_pair(ak, ai, bk, bi):
    lok, loi, hik, hii = _lex_exchange(ak, ai, _rev(bk), _rev(bi))
    lk, li = plsc.sort_key_val(lok, loi)
    hk, hi = plsc.sort_key_val(hik, hii)
    return lk, li, hk, hi


def _merge16(b0k, b0i, b1k, b1i, ck, ci):
    lok, loi, _, _ = _lex_exchange(ck, ci, _rev(b1k), _rev(b1i))
    lk, li = plsc.sort_key_val(lok, loi)
    nlk, nli, nhk, nhi = _lex_exchange(b0k, b0i, _rev(lk), _rev(li))
    b0k, b0i = plsc.sort_key_val(nlk, nli)
    b1k, b1i = plsc.sort_key_val(nhk, nhi)
    return b0k, b0i, b1k, b1i


def _top32(b0k, b0i, b1k, b1i, n0k, n0i, n1k, n1i):
    st = _merge16(b0k, b0i, b1k, b1i, n0k, n0i)
    return _merge16(*st, n1k, n1i)


def _sc_topk(dist2d, thr1d):
    rows = dist2d.shape[0]
    rpw = rows // NWORKERS
    mesh = plsc.VectorSubcoreMesh(core_axis_name="c", subcore_axis_name="s")

    @functools.partial(
        pl.kernel,
        out_type=jax.ShapeDtypeStruct((rows * K,), jnp.int32),
        mesh=mesh,
        compiler_params=pltpu.CompilerParams(needs_layout_passes=False),
        scratch_types=[
            pltpu.VMEM((rpw + 16,), jnp.float32),
            pltpu.VMEM((N + 16,), jnp.float32),
            pltpu.VMEM((N + 16,), jnp.float32),
            pltpu.VMEM((CANDCAP,), jnp.int32),
            pltpu.VMEM((CANDCAP,), jnp.int32),
            pltpu.VMEM((2 * K,), jnp.int32),
            pltpu.VMEM((rpw * K,), jnp.int32),
            pltpu.SemaphoreType.DMA,
            pltpu.SemaphoreType.DMA,
        ],
    )
    def sc_kernel(dist_hbm, thr_hbm, out_hbm, thr_v, row0, row1,
                  candi, candp, pairb, outb, sem0, sem1):
        wid = lax.axis_index("s") * 2 + lax.axis_index("c")
        base = wid * rpw

        pltpu.sync_copy(thr_hbm.at[pl.ds(base, rpw)], thr_v.at[pl.ds(0, rpw)])

        iota = lax.iota(jnp.int32, 16)
        inf16 = jnp.full((16,), jnp.inf, jnp.float32)
        row0[pl.ds(N, 16)] = inf16
        row1[pl.ds(N, 16)] = inf16
        padi16 = jnp.full((16,), N, jnp.int32)

        def issue(r, buf, sem):
            pltpu.make_async_copy(dist_hbm.at[base + r],
                                  buf.at[pl.ds(0, N)], sem).start()

        def wait(r, buf, sem):
            pltpu.make_async_copy(dist_hbm.at[base + r],
                                  buf.at[pl.ds(0, N)], sem).wait()

        lane_base = iota * (N // 16)

        def process(r, buf):
            tb = plsc.load_gather(thr_v, [jnp.full((16,), r, jnp.int32)])

            sixteen = jnp.full((16,), 16, jnp.int32)

            @plsc.parallel_loop(0, N // 16, unroll=8,
                                carry=(lane_base, iota))
            def scan_carry(j, carry):
                pos, colv = carry
                v = buf[pl.ds(j * 16, 16)]
                m = v <= tb
                plsc.store_scatter(candi, [pos], colv, mask=m)
                return (pos + m.astype(jnp.int32), colv + sixteen)

            posf, _ = scan_carry
            percnt = posf - lane_base

            cum = plsc.cumsum(percnt)
            total = cum[15]
            offs = cum - percnt
            pairb[0:16] = percnt
            pairb[16:32] = offs

            @plsc.parallel_loop(0, 16)
            def _move1(l):
                li = jnp.full((16,), l, jnp.int32)
                cl16 = plsc.load_gather(pairb, [li])
                ol16 = plsc.load_gather(pairb, [li + sixteen])
                src = candi[pl.ds(l * (N // 16), 16)]
                plsc.store_scatter(candp, [ol16 + iota], src,
                                   mask=iota < cl16)

            sp, _ = plsc.sort_key_val(percnt, percnt)
            maxcnt = sp[15]

            @pl.when(maxcnt > 16)
            def _slow_compact():
                for l in range(16):
                    cl = percnt[l]
                    ol = cum[l] - cl

                    def move(t, _, l=l, cl=cl, ol=ol):
                        src = candi[pl.ds(l * (N // 16) + t * 16, 16)]
                        mm = (iota + jnp.full((16,), t * 16, jnp.int32)
                              ) < jnp.full((16,), cl, jnp.int32)
                        plsc.store_scatter(
                            candp,
                            [jnp.full((16,), ol + t * 16, jnp.int32) + iota],
                            src, mask=mm)
                        return _

                    lax.fori_loop(0, (cl + 15) // 16, move, jnp.int32(0))

            cnt = total
            candp[pl.ds(cnt, 16)] = padi16
            candp[pl.ds(cnt + 16, 16)] = padi16

            def sorted16(j):
                ci = candp[pl.ds(j * 16, 16)]
                ck = plsc.load_gather(buf, [ci])
                return plsc.sort_key_val(ck, ci)

            ak, ai = sorted16(0)
            bk, bi = sorted16(1)
            b0k, b0i, b1k, b1i = _merge_pair(ak, ai, bk, bi)

            def mbody(p, st):
                ak, ai = sorted16(2 * p + 2)
                ck, ci = sorted16(2 * p + 3)
                n0k, n0i, n1k, n1i = _merge_pair(ak, ai, ck, ci)
                return _top32(*st, n0k, n0i, n1k, n1i)

            nv = (cnt + 15) // 16
            b0k, b0i, b1k, b1i = lax.fori_loop(
                0, (nv - 1) // 2, mbody, (b0k, b0i, b1k, b1i))

            pairb[0:16] = b0i
            pairb[16:32] = b1i
            outv = plsc.load_gather(pairb, [iota * 2])
            outb[pl.ds(r * K, K)] = outv

        issue(0, row0, sem0)
        issue(1, row1, sem1)

        def outer(i, carry):
            r0 = 2 * i
            wait(r0, row0, sem0)
            process(r0, row0)

            @pl.when(r0 + 2 < rpw)
            def _():
                issue(r0 + 2, row0, sem0)

            r1 = 2 * i + 1
            wait(r1, row1, sem1)
            process(r1, row1)

            @pl.when(r1 + 2 < rpw)
            def _():
                issue(r1 + 2, row1, sem1)

            return carry

        lax.fori_loop(0, rpw // 2, outer, jnp.int32(0))

        pltpu.sync_copy(outb, out_hbm.at[pl.ds(base * K, rpw * K)])

    return sc_kernel(dist2d, thr1d)


@jax.jit
def kernel(query):
    outs = []
    for b in range(B):
        dist, thr = _tc_stage(query[b:b + 1])
        outs.append(_sc_topk(dist.reshape(N, N), thr.reshape(N)))
    return jnp.stack(outs).reshape(B, N, K)


# speedup vs baseline: 1.0631x; 1.0631x over previous
import functools

import jax
import jax.numpy as jnp
from jax import lax
from jax.experimental import pallas as pl
from jax.experimental.pallas import tpu as pltpu
from jax.experimental.pallas import tpu_sc as plsc

K = 16
NUM_RANKS = 31

B = 4
N = 4096
C = 256
BM = 256
NCHUNK = 32

NWORKERS = 32
ROWS = B * N
RPW = ROWS // NWORKERS
CANDCAP = N + 32
BIGI = 2**30


def _dist_block(q_ref, qt_ref, dist_ref, thr_ref):
    qb = q_ref[0]
    st = qt_ref[0]

    b2 = jnp.sum(qb * qb, axis=1, keepdims=True)
    a2 = jnp.sum(st * st, axis=0, keepdims=True)
    dot = jax.lax.dot_general(
        qb, st, (((1,), (0,)), ((), ())),
        preferred_element_type=jnp.float32)
    d2 = (a2 + b2) - 2.0 * dot
    dist = jnp.sqrt(jnp.maximum(d2, 1e-12))
    dist_ref[0] = dist

    w = N // NCHUNK
    thr = jnp.min(dist[:, :w], axis=1, keepdims=True)
    for c in range(1, NCHUNK):
        cm = jnp.min(dist[:, c * w:(c + 1) * w], axis=1, keepdims=True)
        thr = jnp.maximum(thr, cm)
    thr_ref[0] = thr


def _tc_stage(query):
    nb = query.shape[0]
    qt = jnp.swapaxes(query, 1, 2)
    return pl.pallas_call(
        _dist_block,
        grid=(nb, N // BM),
        in_specs=[
            pl.BlockSpec((1, BM, C), lambda b, i: (b, i, 0)),
            pl.BlockSpec((1, C, N), lambda b, i: (b, 0, 0)),
        ],
        out_specs=[
            pl.BlockSpec((1, BM, N), lambda b, i: (b, i, 0)),
            pl.BlockSpec((1, BM, 1), lambda b, i: (b, i, 0)),
        ],
        out_shape=[
            jax.ShapeDtypeStruct((nb, N, N), jnp.float32),
            jax.ShapeDtypeStruct((nb, N, 1), jnp.float32),
        ],
    )(query, qt)


def _lex_exchange(ak, ai, bk, bi):
    t = (ak < bk) | ((ak == bk) & (ai < bi))
    lok = jnp.where(t, ak, bk)
    loi = jnp.where(t, ai, bi)
    hik = jnp.where(t, bk, ak)
    hii = jnp.where(t, bi, ai)
    return lok, loi, hik, hii


def _rev(x):
    return lax.rev(x, (0,))


def _lex_min(ak, ai, bk, bi):
    t = (ak < bk) | ((ak == bk) & (ai < bi))
    return jnp.where(t, ak, bk), jnp.where(t, ai, bi)


def _top32_desc(b0k, b0i, b1k, b1i, rn0k, rn0i, rn1k, rn1i):
    c0k, c0i = _lex_min(b0k, b0i, rn1k, rn1i)
    c1k, c1i = _lex_min(b1k, b1i, rn0k, rn0i)
    lok, loi, hik, hii = _lex_exchange(c0k, c0i, c1k, c1i)
    b0k, b0i = plsc.sort_key_val(lok, loi)
    b1k, b1i = plsc.sort_key_val(hik, hii)
    return b0k, b0i, b1k, b1i


def _sc_topk(dist2d, thr1d):
    rows = dist2d.shape[0]
    rpw = rows // NWORKERS
    mesh = plsc.VectorSubcoreMesh(core_axis_name="c", subcore_axis_name="s")

    @functools.partial(
        pl.kernel,
        out_type=jax.ShapeDtypeStruct((rows * K,), jnp.int32),
        mesh=mesh,
        compiler_params=pltpu.CompilerParams(needs_layout_passes=False),
        scratch_types=[
            pltpu.VMEM((rpw + 16,), jnp.float32),
            pltpu.VMEM((N + 16,), jnp.float32),
            pltpu.VMEM((N + 16,), jnp.float32),
            pltpu.VMEM((CANDCAP,), jnp.int32),
            pltpu.VMEM((CANDCAP,), jnp.int32),
            pltpu.VMEM((2 * K,), jnp.int32),
            pltpu.VMEM((rpw * K,), jnp.int32),
            pltpu.SemaphoreType.DMA,
            pltpu.SemaphoreType.DMA,
        ],
    )
    def sc_kernel(dist_hbm, thr_hbm, out_hbm, thr_v, row0, row1,
                  candi, candp, pairb, outb, sem0, sem1):
        wid = lax.axis_index("s") * 2 + lax.axis_index("c")
        base = wid * rpw

        pltpu.sync_copy(thr_hbm.at[pl.ds(base, rpw)], thr_v.at[pl.ds(0, rpw)])

        iota = lax.iota(jnp.int32, 16)
        inf16 = jnp.full((16,), jnp.inf, jnp.float32)
        row0[pl.ds(N, 16)] = inf16
        row1[pl.ds(N, 16)] = inf16
        padi16 = jnp.full((16,), N, jnp.int32)

        def issue(r, buf, sem):
            pltpu.make_async_copy(dist_hbm.at[base + r],
                                  buf.at[pl.ds(0, N)], sem).start()

        def wait(r, buf, sem):
            pltpu.make_async_copy(dist_hbm.at[base + r],
                                  buf.at[pl.ds(0, N)], sem).wait()

        lane_base = iota * (N // 16)

        def process(r, buf):
            tb = plsc.load_gather(thr_v, [jnp.full((16,), r, jnp.int32)])

            sixteen = jnp.full((16,), 16, jnp.int32)

            @plsc.parallel_loop(0, N // 16, unroll=8,
                                carry=(lane_base, iota))
            def scan_carry(j, carry):
                pos, colv = carry
                v = buf[pl.ds(j * 16, 16)]
                m = v <= tb
                plsc.store_scatter(candi, [pos], colv, mask=m)
                return (pos + m.astype(jnp.int32), colv + sixteen)

            posf, _ = scan_carry
            percnt = posf - lane_base

            cum = plsc.cumsum(percnt)
            total = cum[15]
            offs = cum - percnt
            pairb[0:16] = percnt
            pairb[16:32] = offs

            @plsc.parallel_loop(0, 16)
            def _move1(l):
                li = jnp.full((16,), l, jnp.int32)
                cl16 = plsc.load_gather(pairb, [li])
                ol16 = plsc.load_gather(pairb, [li + sixteen])
                src = candi[pl.ds(l * (N // 16), 16)]
                plsc.store_scatter(candp, [ol16 + iota], src,
                                   mask=iota < cl16)

            sp, _ = plsc.sort_key_val(percnt, percnt)
            maxcnt = sp[15]

            @pl.when(maxcnt > 16)
            def _slow_compact():
                for l in range(16):
                    cl = percnt[l]
                    ol = cum[l] - cl

                    def move(t, _, l=l, cl=cl, ol=ol):
                        src = candi[pl.ds(l * (N // 16) + t * 16, 16)]
                        mm = (iota + jnp.full((16,), t * 16, jnp.int32)
                              ) < jnp.full((16,), cl, jnp.int32)
                        plsc.store_scatter(
                            candp,
                            [jnp.full((16,), ol + t * 16, jnp.int32) + iota],
                            src, mask=mm)
                        return _

                    lax.fori_loop(0, (cl + 15) // 16, move, jnp.int32(0))

            cnt = total
            candp[pl.ds(cnt, 16)] = padi16
            candp[pl.ds(cnt + 16, 16)] = padi16

            def sorted16(j, descending=False):
                ci = candp[pl.ds(j * 16, 16)]
                ck = plsc.load_gather(buf, [ci])
                return plsc.sort_key_val(ck, ci, descending=descending)

            ak, ai = sorted16(0)
            bk, bi = sorted16(1, descending=True)
            lok, loi, hik, hii = _lex_exchange(ak, ai, bk, bi)
            b0k, b0i = plsc.sort_key_val(lok, loi)
            b1k, b1i = plsc.sort_key_val(hik, hii)

            def mbody(p, st):
                ak, ai = sorted16(2 * p + 2)
                ck, ci = sorted16(2 * p + 3, descending=True)
                lok, loi, hik, hii = _lex_exchange(ak, ai, ck, ci)
                rn0k, rn0i = plsc.sort_key_val(lok, loi, descending=True)
                rn1k, rn1i = plsc.sort_key_val(hik, hii, descending=True)
                return _top32_desc(*st, rn0k, rn0i, rn1k, rn1i)

            nv = (cnt + 15) // 16
            b0k, b0i, b1k, b1i = lax.fori_loop(
                0, (nv - 1) // 2, mbody, (b0k, b0i, b1k, b1i))

            pairb[0:16] = b0i
            pairb[16:32] = b1i
            outv = plsc.load_gather(pairb, [iota * 2])
            outb[pl.ds(r * K, K)] = outv

        issue(0, row0, sem0)
        issue(1, row1, sem1)

        def outer(i, carry):
            r0 = 2 * i
            wait(r0, row0, sem0)
            process(r0, row0)

            @pl.when(r0 + 2 < rpw)
            def _():
                issue(r0 + 2, row0, sem0)

            r1 = 2 * i + 1
            wait(r1, row1, sem1)
            process(r1, row1)

            @pl.when(r1 + 2 < rpw)
            def _():
                issue(r1 + 2, row1, sem1)

            return carry

        lax.fori_loop(0, rpw // 2, outer, jnp.int32(0))

        pltpu.sync_copy(outb, out_hbm.at[pl.ds(base * K, rpw * K)])

    return sc_kernel(dist2d, thr1d)


@jax.jit
def kernel(query):
    outs = []
    for b in range(B):
        dist, thr = _tc_stage(query[b:b + 1])
        outs.append(_sc_topk(dist.reshape(N, N), thr.reshape(N)))
    return jnp.stack(outs).reshape(B, N, K)


# speedup vs baseline: 1.0643x; 1.0011x over previous
import functools

import jax
import jax.numpy as jnp
from jax import lax
from jax.experimental import pallas as pl
from jax.experimental.pallas import tpu as pltpu
from jax.experimental.pallas import tpu_sc as plsc

K = 16

B = 4
N = 4096
C = 256
BM = 256
NCHUNK = 32

NWORKERS = 32
CANDCAP = N + 32


def _dist_block(q_ref, qt_ref, dist_ref, thr_ref):
    qb = q_ref[0]
    st = qt_ref[0]

    b2 = jnp.sum(qb * qb, axis=1, keepdims=True)
    a2 = jnp.sum(st * st, axis=0, keepdims=True)
    dot = jax.lax.dot_general(
        qb, st, (((1,), (0,)), ((), ())),
        preferred_element_type=jnp.float32)
    d2 = (a2 + b2) - 2.0 * dot
    dist = jnp.sqrt(jnp.maximum(d2, 1e-12))
    dist_ref[0] = dist

    w = N // NCHUNK
    thr = jnp.min(dist[:, :w], axis=1, keepdims=True)
    for c in range(1, NCHUNK):
        cm = jnp.min(dist[:, c * w:(c + 1) * w], axis=1, keepdims=True)
        thr = jnp.maximum(thr, cm)
    thr_ref[0] = thr


def _tc_stage(query):
    nb = query.shape[0]
    qt = jnp.swapaxes(query, 1, 2)
    return pl.pallas_call(
        _dist_block,
        grid=(nb, N // BM),
        in_specs=[
            pl.BlockSpec((1, BM, C), lambda b, i: (b, i, 0)),
            pl.BlockSpec((1, C, N), lambda b, i: (b, 0, 0)),
        ],
        out_specs=[
            pl.BlockSpec((1, BM, N), lambda b, i: (b, i, 0)),
            pl.BlockSpec((1, BM, 1), lambda b, i: (b, i, 0)),
        ],
        out_shape=[
            jax.ShapeDtypeStruct((nb, N, N), jnp.float32),
            jax.ShapeDtypeStruct((nb, N, 1), jnp.float32),
        ],
    )(query, qt)


def _lex_exchange(ak, ai, bk, bi):
    t = (ak < bk) | ((ak == bk) & (ai < bi))
    lok = jnp.where(t, ak, bk)
    loi = jnp.where(t, ai, bi)
    hik = jnp.where(t, bk, ak)
    hii = jnp.where(t, bi, ai)
    return lok, loi, hik, hii


def _lex_min(ak, ai, bk, bi):
    t = (ak < bk) | ((ak == bk) & (ai < bi))
    return jnp.where(t, ak, bk), jnp.where(t, ai, bi)


def _top32_desc(b0k, b0i, b1k, b1i, rn0k, rn0i, rn1k, rn1i):
    c0k, c0i = _lex_min(b0k, b0i, rn1k, rn1i)
    c1k, c1i = _lex_min(b1k, b1i, rn0k, rn0i)
    lok, loi, hik, hii = _lex_exchange(c0k, c0i, c1k, c1i)
    b0k, b0i = plsc.sort_key_val(lok, loi)
    b1k, b1i = plsc.sort_key_val(hik, hii)
    return b0k, b0i, b1k, b1i


def _sc_topk(dist2d, thr1d):
    rows = dist2d.shape[0]
    rpw = rows // NWORKERS
    mesh = plsc.VectorSubcoreMesh(core_axis_name="c", subcore_axis_name="s")

    @functools.partial(
        pl.kernel,
        out_type=jax.ShapeDtypeStruct((rows * K,), jnp.int32),
        mesh=mesh,
        compiler_params=pltpu.CompilerParams(needs_layout_passes=False),
        scratch_types=[
            pltpu.VMEM((rpw + 16,), jnp.float32),
            pltpu.VMEM((N + 16,), jnp.float32),
            pltpu.VMEM((N + 16,), jnp.float32),
            pltpu.VMEM((CANDCAP,), jnp.int32),
            pltpu.VMEM((CANDCAP,), jnp.int32),
            pltpu.VMEM((2 * K,), jnp.int32),
            pltpu.VMEM((rpw * K,), jnp.int32),
            pltpu.SemaphoreType.DMA,
            pltpu.SemaphoreType.DMA,
        ],
    )
    def sc_kernel(dist_hbm, thr_hbm, out_hbm, thr_v, row0, row1,
                  candi, candp, pairb, outb, sem0, sem1):
        wid = lax.axis_index("s") * 2 + lax.axis_index("c")
        base = wid * rpw

        pltpu.sync_copy(thr_hbm.at[pl.ds(base, rpw)], thr_v.at[pl.ds(0, rpw)])

        iota = lax.iota(jnp.int32, 16)
        inf16 = jnp.full((16,), jnp.inf, jnp.float32)
        row0[pl.ds(N, 16)] = inf16
        row1[pl.ds(N, 16)] = inf16
        padi16 = jnp.full((16,), N, jnp.int32)

        def issue(r, buf, sem):
            pltpu.make_async_copy(dist_hbm.at[base + r],
                                  buf.at[pl.ds(0, N)], sem).start()

        def wait(r, buf, sem):
            pltpu.make_async_copy(dist_hbm.at[base + r],
                                  buf.at[pl.ds(0, N)], sem).wait()

        lane_base = iota * (N // 16)

        def process(r, buf):
            tb = plsc.load_gather(thr_v, [jnp.full((16,), r, jnp.int32)])

            sixteen = jnp.full((16,), 16, jnp.int32)

            @plsc.parallel_loop(0, N // 16, unroll=8,
                                carry=(lane_base, iota))
            def scan_carry(j, carry):
                pos, colv = carry
                v = buf[pl.ds(j * 16, 16)]
                m = v <= tb
                plsc.store_scatter(candi, [pos], colv, mask=m)
                return (pos + m.astype(jnp.int32), colv + sixteen)

            posf, _ = scan_carry
            percnt = posf - lane_base

            cum = plsc.cumsum(percnt)
            total = cum[15]
            offs = cum - percnt
            pairb[0:16] = percnt
            pairb[16:32] = offs

            @plsc.parallel_loop(0, 16)
            def _move1(l):
                li = jnp.full((16,), l, jnp.int32)
                cl16 = plsc.load_gather(pairb, [li])
                ol16 = plsc.load_gather(pairb, [li + sixteen])
                src = candi[pl.ds(l * (N // 16), 16)]
                plsc.store_scatter(candp, [ol16 + iota], src,
                                   mask=iota < cl16)

            sp, _ = plsc.sort_key_val(percnt, percnt)
            maxcnt = sp[15]

            @pl.when(maxcnt > 16)
            def _slow_compact():
                for l in range(16):
                    cl = percnt[l]
                    ol = cum[l] - cl

                    def move(t, _, l=l, cl=cl, ol=ol):
                        src = candi[pl.ds(l * (N // 16) + t * 16, 16)]
                        mm = (iota + jnp.full((16,), t * 16, jnp.int32)
                              ) < jnp.full((16,), cl, jnp.int32)
                        plsc.store_scatter(
                            candp,
                            [jnp.full((16,), ol + t * 16, jnp.int32) + iota],
                            src, mask=mm)
                        return _

                    lax.fori_loop(0, (cl + 15) // 16, move, jnp.int32(0))

            cnt = total
            candp[pl.ds(cnt, 16)] = padi16
            candp[pl.ds(cnt + 16, 16)] = padi16

            def sorted16(j, descending=False):
                ci = candp[pl.ds(j * 16, 16)]
                ck = plsc.load_gather(buf, [ci])
                return plsc.sort_key_val(ck, ci, descending=descending)

            ak, ai = sorted16(0)
            bk, bi = sorted16(1, descending=True)
            lok, loi, hik, hii = _lex_exchange(ak, ai, bk, bi)
            b0k, b0i = plsc.sort_key_val(lok, loi)
            b1k, b1i = plsc.sort_key_val(hik, hii)

            def mbody(p, st):
                ak, ai = sorted16(2 * p + 2)
                ck, ci = sorted16(2 * p + 3, descending=True)
                lok, loi, hik, hii = _lex_exchange(ak, ai, ck, ci)
                rn0k, rn0i = plsc.sort_key_val(lok, loi, descending=True)
                rn1k, rn1i = plsc.sort_key_val(hik, hii, descending=True)
                return _top32_desc(*st, rn0k, rn0i, rn1k, rn1i)

            nv = (cnt + 15) // 16
            b0k, b0i, b1k, b1i = lax.fori_loop(
                0, (nv - 1) // 2, mbody, (b0k, b0i, b1k, b1i))

            pairb[0:16] = b0i
            pairb[16:32] = b1i
            outv = plsc.load_gather(pairb, [iota * 2])
            outb[pl.ds(r * K, K)] = outv

        issue(0, row0, sem0)
        issue(1, row1, sem1)

        def outer(i, carry):
            r0 = 2 * i
            wait(r0, row0, sem0)
            process(r0, row0)

            @pl.when(r0 + 2 < rpw)
            def _():
                issue(r0 + 2, row0, sem0)

            r1 = 2 * i + 1
            wait(r1, row1, sem1)
            process(r1, row1)

            @pl.when(r1 + 2 < rpw)
            def _():
                issue(r1 + 2, row1, sem1)

            return carry

        lax.fori_loop(0, rpw // 2, outer, jnp.int32(0))

        pltpu.sync_copy(outb, out_hbm.at[pl.ds(base * K, rpw * K)])

    return sc_kernel(dist2d, thr1d)


@jax.jit
def kernel(query):
    outs = []
    for b in range(B):
        dist, thr = _tc_stage(query[b:b + 1])
        outs.append(_sc_topk(dist.reshape(N, N), thr.reshape(N)))
    return jnp.stack(outs).reshape(B, N, K)


# speedup vs baseline: 1.1135x; 1.0462x over previous
import functools

import jax
import jax.numpy as jnp
from jax import lax
from jax.experimental import pallas as pl
from jax.experimental.pallas import tpu as pltpu
from jax.experimental.pallas import tpu_sc as plsc

K = 16

B = 4
N = 4096
C = 256
BM = 256
NCHUNK = 32

NWORKERS = 32
CANDCAP = N + 32


def _dist_block(q_ref, qt_ref, dist_ref, thr_ref):
    qb = q_ref[0]
    st = qt_ref[0]

    b2 = jnp.sum(qb * qb, axis=1, keepdims=True)
    a2 = jnp.sum(st * st, axis=0, keepdims=True)
    dot = jax.lax.dot_general(
        qb, st, (((1,), (0,)), ((), ())),
        preferred_element_type=jnp.float32)
    d2 = (a2 + b2) - 2.0 * dot
    dist = jnp.sqrt(jnp.maximum(d2, 1e-12))
    dist_ref[0] = dist

    w = N // NCHUNK
    t1 = jnp.min(dist[:, :w], axis=1, keepdims=True)
    t2 = jnp.full_like(t1, -jnp.inf)
    for c in range(1, NCHUNK):
        cm = jnp.min(dist[:, c * w:(c + 1) * w], axis=1, keepdims=True)
        t2 = jnp.maximum(t2, jnp.minimum(t1, cm))
        t1 = jnp.maximum(t1, cm)
    thr_ref[0] = t2


def _tc_stage(query):
    nb = query.shape[0]
    qt = jnp.swapaxes(query, 1, 2)
    return pl.pallas_call(
        _dist_block,
        grid=(nb, N // BM),
        in_specs=[
            pl.BlockSpec((1, BM, C), lambda b, i: (b, i, 0)),
            pl.BlockSpec((1, C, N), lambda b, i: (b, 0, 0)),
        ],
        out_specs=[
            pl.BlockSpec((1, BM, N), lambda b, i: (b, i, 0)),
            pl.BlockSpec((1, BM, 1), lambda b, i: (b, i, 0)),
        ],
        out_shape=[
            jax.ShapeDtypeStruct((nb, N, N), jnp.float32),
            jax.ShapeDtypeStruct((nb, N, 1), jnp.float32),
        ],
    )(query, qt)


def _lex_exchange(ak, ai, bk, bi):
    t = (ak < bk) | ((ak == bk) & (ai < bi))
    lok = jnp.where(t, ak, bk)
    loi = jnp.where(t, ai, bi)
    hik = jnp.where(t, bk, ak)
    hii = jnp.where(t, bi, ai)
    return lok, loi, hik, hii


def _lex_min(ak, ai, bk, bi):
    t = (ak < bk) | ((ak == bk) & (ai < bi))
    return jnp.where(t, ak, bk), jnp.where(t, ai, bi)


def _top32_desc(b0k, b0i, b1k, b1i, rn0k, rn0i, rn1k, rn1i):
    c0k, c0i = _lex_min(b0k, b0i, rn1k, rn1i)
    c1k, c1i = _lex_min(b1k, b1i, rn0k, rn0i)
    lok, loi, hik, hii = _lex_exchange(c0k, c0i, c1k, c1i)
    b0k, b0i = plsc.sort_key_val(lok, loi)
    b1k, b1i = plsc.sort_key_val(hik, hii)
    return b0k, b0i, b1k, b1i


def _sc_topk(dist2d, thr1d):
    rows = dist2d.shape[0]
    rpw = rows // NWORKERS
    mesh = plsc.VectorSubcoreMesh(core_axis_name="c", subcore_axis_name="s")

    @functools.partial(
        pl.kernel,
        out_type=jax.ShapeDtypeStruct((rows * K,), jnp.int32),
        mesh=mesh,
        compiler_params=pltpu.CompilerParams(needs_layout_passes=False),
        scratch_types=[
            pltpu.VMEM((rpw + 16,), jnp.float32),
            pltpu.VMEM((N + 16,), jnp.float32),
            pltpu.VMEM((N + 16,), jnp.float32),
            pltpu.VMEM((CANDCAP,), jnp.int32),
            pltpu.VMEM((CANDCAP,), jnp.int32),
            pltpu.VMEM((2 * K,), jnp.int32),
            pltpu.VMEM((rpw * K,), jnp.int32),
            pltpu.SemaphoreType.DMA,
            pltpu.SemaphoreType.DMA,
        ],
    )
    def sc_kernel(dist_hbm, thr_hbm, out_hbm, thr_v, row0, row1,
                  candi, candp, pairb, outb, sem0, sem1):
        wid = lax.axis_index("s") * 2 + lax.axis_index("c")
        base = wid * rpw

        pltpu.sync_copy(thr_hbm.at[pl.ds(base, rpw)], thr_v.at[pl.ds(0, rpw)])

        iota = lax.iota(jnp.int32, 16)
        inf16 = jnp.full((16,), jnp.inf, jnp.float32)
        row0[pl.ds(N, 16)] = inf16
        row1[pl.ds(N, 16)] = inf16
        padi16 = jnp.full((16,), N, jnp.int32)

        def issue(r, buf, sem):
            pltpu.make_async_copy(dist_hbm.at[base + r],
                                  buf.at[pl.ds(0, N)], sem).start()

        def wait(r, buf, sem):
            pltpu.make_async_copy(dist_hbm.at[base + r],
                                  buf.at[pl.ds(0, N)], sem).wait()

        lane_base = iota * (N // 16)

        def process(r, buf):
            tb = plsc.load_gather(thr_v, [jnp.full((16,), r, jnp.int32)])

            sixteen = jnp.full((16,), 16, jnp.int32)

            @plsc.parallel_loop(0, N // 16, unroll=8,
                                carry=(lane_base, iota))
            def scan_carry(j, carry):
                pos, colv = carry
                v = buf[pl.ds(j * 16, 16)]
                m = v <= tb
                plsc.store_scatter(candi, [pos], colv, mask=m)
                return (pos + m.astype(jnp.int32), colv + sixteen)

            posf, _ = scan_carry
            percnt = posf - lane_base

            cum = plsc.cumsum(percnt)
            total = cum[15]
            offs = cum - percnt
            pairb[0:16] = percnt
            pairb[16:32] = offs

            @plsc.parallel_loop(0, 16)
            def _move1(l):
                li = jnp.full((16,), l, jnp.int32)
                cl16 = plsc.load_gather(pairb, [li])
                ol16 = plsc.load_gather(pairb, [li + sixteen])
                src = candi[pl.ds(l * (N // 16), 16)]
                plsc.store_scatter(candp, [ol16 + iota], src,
                                   mask=iota < cl16)

            sp, _ = plsc.sort_key_val(percnt, percnt)
            maxcnt = sp[15]

            @pl.when(maxcnt > 16)
            def _slow_compact():
                for l in range(16):
                    cl = percnt[l]
                    ol = cum[l] - cl

                    def move(t, _, l=l, cl=cl, ol=ol):
                        src = candi[pl.ds(l * (N // 16) + t * 16, 16)]
                        mm = (iota + jnp.full((16,), t * 16, jnp.int32)
                              ) < jnp.full((16,), cl, jnp.int32)
                        plsc.store_scatter(
                            candp,
                            [jnp.full((16,), ol + t * 16, jnp.int32) + iota],
                            src, mask=mm)
                        return _

                    lax.fori_loop(0, (cl + 15) // 16, move, jnp.int32(0))

            cnt = total
            candp[pl.ds(cnt, 16)] = padi16
            candp[pl.ds(cnt + 16, 16)] = padi16

            def sorted16(j, descending=False):
                ci = candp[pl.ds(j * 16, 16)]
                ck = plsc.load_gather(buf, [ci])
                return plsc.sort_key_val(ck, ci, descending=descending)

            ak, ai = sorted16(0)
            bk, bi = sorted16(1, descending=True)
            lok, loi, hik, hii = _lex_exchange(ak, ai, bk, bi)
            b0k, b0i = plsc.sort_key_val(lok, loi)
            b1k, b1i = plsc.sort_key_val(hik, hii)

            def mbody(p, st):
                ak, ai = sorted16(2 * p + 2)
                ck, ci = sorted16(2 * p + 3, descending=True)
                lok, loi, hik, hii = _lex_exchange(ak, ai, ck, ci)
                rn0k, rn0i = plsc.sort_key_val(lok, loi, descending=True)
                rn1k, rn1i = plsc.sort_key_val(hik, hii, descending=True)
                return _top32_desc(*st, rn0k, rn0i, rn1k, rn1i)

            nv = (cnt + 15) // 16
            b0k, b0i, b1k, b1i = lax.fori_loop(
                0, (nv - 1) // 2, mbody, (b0k, b0i, b1k, b1i))

            pairb[0:16] = b0i
            pairb[16:32] = b1i
            outv = plsc.load_gather(pairb, [iota * 2])
            outb[pl.ds(r * K, K)] = outv

        issue(0, row0, sem0)
        issue(1, row1, sem1)

        def outer(i, carry):
            r0 = 2 * i
            wait(r0, row0, sem0)
            process(r0, row0)

            @pl.when(r0 + 2 < rpw)
            def _():
                issue(r0 + 2, row0, sem0)

            r1 = 2 * i + 1
            wait(r1, row1, sem1)
            process(r1, row1)

            @pl.when(r1 + 2 < rpw)
            def _():
                issue(r1 + 2, row1, sem1)

            return carry

        lax.fori_loop(0, rpw // 2, outer, jnp.int32(0))

        pltpu.sync_copy(outb, out_hbm.at[pl.ds(base * K, rpw * K)])

    return sc_kernel(dist2d, thr1d)


@jax.jit
def kernel(query):
    outs = []
    for b in range(B):
        dist, thr = _tc_stage(query[b:b + 1])
        outs.append(_sc_topk(dist.reshape(N, N), thr.reshape(N)))
    return jnp.stack(outs).reshape(B, N, K)
